# baseline (speedup 1.0000x reference)
import functools

import jax
import jax.numpy as jnp
from jax import lax
from jax.experimental import pallas as pl
from jax.experimental.pallas import tpu as pltpu
from jax.experimental.pallas import tpu_sc as plsc

_N = 10000
_K = 16
_D = 256
_H = 256
_HW = _H // 2
_LANES = 16
_NTILES = 16
_CHUNK = 8
_CK = _CHUNK * _K
_NBUF = 4
_NP = 10240
_NPT0 = 512
_NPT1 = 128
_NC0 = _NPT0 // _CHUNK
_NC1 = _NPT1 // _CHUNK
_NE = _NP * _K + (_NPT0 - _NPT1) * _K
_NG2 = _H // (2 * _LANES)
_HIMASK = -65536


def _pack_words(v):
    f = v.shape[-1]
    lo = v[:, :f // 2].astype(jnp.bfloat16).astype(jnp.float32)
    hi = v[:, f // 2:].astype(jnp.bfloat16).astype(jnp.float32)
    lo_u = lax.shift_right_logical(
        lax.bitcast_convert_type(lo, jnp.int32), 16)
    hi_u = lax.bitcast_convert_type(hi, jnp.int32) & _HIMASK
    return lo_u | hi_u


def _unpack_words(w):
    lo = lax.bitcast_convert_type(lax.shift_left(w, 16), jnp.float32)
    hi = lax.bitcast_convert_type(w & _HIMASK, jnp.float32)
    return lo, hi


def _mm_relu_packed(a_packed, a_plain, w, b):
    if a_packed is not None:
        m = a_packed.shape[0]
        kd = a_packed.shape[1] * 2
    else:
        m, kd = a_plain.shape
    hd = w.shape[1]
    bm = 1000

    def body(a_ref, w_ref, b_ref, o_ref):
        if a_packed is not None:
            lo, hi = _unpack_words(a_ref[...])
            acc = jnp.dot(lo, w_ref[:kd // 2],
                          preferred_element_type=jnp.float32)
            acc = acc + jnp.dot(hi, w_ref[kd // 2:],
                                preferred_element_type=jnp.float32)
        else:
            acc = jnp.dot(a_ref[...], w_ref[...],
                          preferred_element_type=jnp.float32)
        o_ref[...] = _pack_words(jnp.maximum(acc + b_ref[...], 0.0))

    a = a_packed if a_packed is not None else a_plain
    return pl.pallas_call(
        body,
        grid=(_N // bm,),
        in_specs=[
            pl.BlockSpec((bm, a.shape[1]), lambda i: (i, 0)),
            pl.BlockSpec((kd, hd), lambda i: (0, 0)),
            pl.BlockSpec((1, hd), lambda i: (0, 0)),
        ],
        out_specs=pl.BlockSpec((bm, hd // 2), lambda i: (i, 0)),
        out_shape=jax.ShapeDtypeStruct((_NP, hd // 2), jnp.int32),
    )(a, w, b.reshape(1, hd))


def _assemble(f1p, f2p, x):
    bm = 1000

    def body(f1_ref, f2_ref, x_ref, o_ref):
        m1, x1 = _unpack_words(f1_ref[...])
        m2, x2 = _unpack_words(f2_ref[...])
        o_ref[:, 0:_H] = m1
        o_ref[:, _H:2 * _H] = x1
        o_ref[:, 2 * _H:3 * _H] = m2
        o_ref[:, 3 * _H:4 * _H] = x2
        o_ref[:, 4 * _H:] = x_ref[...]

    return pl.pallas_call(
        body,
        grid=(_N // bm,),
        in_specs=[
            pl.BlockSpec((bm, _H), lambda i: (i, 0)),
            pl.BlockSpec((bm, _H), lambda i: (i, 0)),
            pl.BlockSpec((bm, _D), lambda i: (i, 0)),
        ],
        out_specs=pl.BlockSpec((bm, 4 * _H + _D), lambda i: (i, 0)),
        out_shape=jax.ShapeDtypeStruct((_N, 4 * _H + _D), jnp.float32),
    )(f1p, f2p, x)


def _tree(vals, op):
    while len(vals) > 1:
        vals = [op(vals[i], vals[i + 1]) for i in range(0, len(vals) - 1, 2)] \
            + ([vals[-1]] if len(vals) % 2 else [])
    return vals[0]


def _sc_acc_body(feat_hbm, idx_hbm, dsq_hbm, out_hbm, *scr):
    idx_all, w_all = scr[0], scr[1]
    rows = list(scr[2:2 + _NBUF])
    own = list(scr[2 + _NBUF:2 + 2 * _NBUF])
    out = list(scr[2 + 2 * _NBUF:2 + 3 * _NBUF])
    gsem = list(scr[2 + 3 * _NBUF:2 + 4 * _NBUF])
    osem = list(scr[2 + 4 * _NBUF:2 + 5 * _NBUF])
    ssem = list(scr[2 + 5 * _NBUF:2 + 6 * _NBUF])

    cid = lax.axis_index("c")
    sid = lax.axis_index("s")
    base = jnp.where(cid == 0, sid * _NPT0,
                     _NTILES * _NPT0 + sid * _NPT1)
    base = pl.multiple_of(base, 16)
    nchunks = jnp.where(cid == 0, _NC0, _NC1)

    def gather_start(ci, b):
        pltpu.async_copy(
            feat_hbm.at[idx_all.at[pl.ds(ci * _CK, _CK)]], rows[b], gsem[b])

    def gather_wait(b):
        pltpu.make_async_copy(
            feat_hbm.at[idx_all.at[pl.ds(0, _CK)]], rows[b], gsem[b]).wait()

    def row0(ci):
        return pl.multiple_of(base + ci * _CHUNK, _CHUNK)

    def own_start(ci, b):
        pltpu.async_copy(
            feat_hbm.at[pl.ds(row0(ci), _CHUNK)], own[b], osem[b])

    def own_wait(b):
        pltpu.make_async_copy(
            feat_hbm.at[pl.ds(0, _CHUNK)], own[b], osem[b]).wait()

    def store_start(ci, b):
        pltpu.async_copy(
            out[b], out_hbm.at[pl.ds(row0(ci), _CHUNK)], ssem[b])

    def store_wait(b):
        pltpu.make_async_copy(
            out[b], out_hbm.at[pl.ds(0, _CHUNK)], ssem[b]).wait()

    pltpu.sync_copy(dsq_hbm.at[pl.ds(base * _K, _NPT0 * _K)], w_all)
    pltpu.sync_copy(idx_hbm.at[pl.ds(base * _K, _NPT0 * _K)], idx_all)
    for b in range(_NBUF):
        gather_start(b, b)
        own_start(b, b)

    def expbody(j, c):
        sl = pl.ds(j * _LANES, _LANES)
        w_all[sl] = jnp.exp(w_all[sl] * -10.0)
        return c

    lax.fori_loop(0, _NPT0 * _K // _LANES, expbody, 0)

    def compute(ci, b):
        rbuf = rows[b]
        obuf = own[b]
        ubuf = out[b]

        def node(n, c):
            wrow = w_all[pl.ds((ci * _CHUNK + n) * _K, _K)]
            dnums = lax.GatherDimensionNumbers(
                offset_dims=(), collapsed_slice_dims=(0,),
                start_index_map=(0,))
            wk = [lax.gather(wrow, jnp.full((_LANES, 1), k, jnp.int32),
                             dnums, slice_sizes=(1,),
                             mode=lax.GatherScatterMode.PROMISE_IN_BOUNDS)
                  for k in range(_K)]
            rbase = n * _K
            for g in range(_NG2):
                col = g * _LANES
                pa = []
                pb = []
                for k in range(_K):
                    va, vb = plsc.unpack(
                        plsc.bitcast(rbuf[rbase + k, pl.ds(col, _LANES)],
                                     jnp.bfloat16),
                        format=plsc.PackFormat.INTERLEAVED)
                    pa.append(va * wk[k])
                    pb.append(vb * wk[k])
                sa = _tree(pa, lambda x, y: x + y)
                sb = _tree(pb, lambda x, y: x + y)
                ma = _tree(pa, jnp.maximum)
                mb = _tree(pb, jnp.maximum)
                oa, ob = plsc.unpack(
                    plsc.bitcast(obuf[n, pl.ds(col, _LANES)], jnp.bfloat16),
                    format=plsc.PackFormat.INTERLEAVED)
                wlo = plsc.pack(sa * (1.0 / _K) - oa, ma - oa,
                                format=plsc.PackFormat.INTERLEAVED)
                whi = plsc.pack(sb * (1.0 / _K) - ob, mb - ob,
                                format=plsc.PackFormat.INTERLEAVED)
                ubuf[n, pl.ds(col, _LANES)] = plsc.bitcast(wlo, jnp.int32)
                ubuf[n, pl.ds(_HW + col, _LANES)] = plsc.bitcast(
                    whi, jnp.int32)
            return c

        lax.fori_loop(0, _CHUNK, node, 0)

    def group(i, c):
        for b in range(_NBUF):
            cch = i * _NBUF + b
            gather_wait(b)
            own_wait(b)

            @pl.when(i > 0)
            def _():
                store_wait(b)

            compute(cch, b)

            @pl.when(cch + _NBUF < nchunks)
            def _():
                gather_start(cch + _NBUF, b)
                own_start(cch + _NBUF, b)

            store_start(cch, b)
        return c

    lax.fori_loop(0, nchunks // _NBUF, group, 0)

    for b in range(_NBUF):
        store_wait(b)


_sc_acc = functools.partial(
    pl.kernel,
    out_type=jax.ShapeDtypeStruct((_NP, _H), jnp.int32),
    mesh=plsc.VectorSubcoreMesh(core_axis_name="c", subcore_axis_name="s",
                                num_cores=2, num_subcores=16),
    compiler_params=pltpu.CompilerParams(needs_layout_passes=False),
    scratch_types=(
        [pltpu.VMEM((_NPT0 * _K,), jnp.int32),
         pltpu.VMEM((_NPT0 * _K,), jnp.float32)]
        + [pltpu.VMEM((_CK, _HW), jnp.int32) for _ in range(_NBUF)]
        + [pltpu.VMEM((_CHUNK, _HW), jnp.int32) for _ in range(_NBUF)]
        + [pltpu.VMEM((_CHUNK, _H), jnp.int32) for _ in range(_NBUF)]
        + [pltpu.SemaphoreType.DMA for _ in range(3 * _NBUF)]
    ),
)(_sc_acc_body)


def kernel(x, neighbor_indices, distancesq, W0, b0, W1, b1):
    idxp = jnp.pad(neighbor_indices.reshape(-1), (0, _NE - _N * _K))
    dsqp = jnp.pad(distancesq.reshape(-1), (0, _NE - _N * _K))
    f0 = _mm_relu_packed(None, x, W0, b0)
    f1p = _sc_acc(f0, idxp, dsqp)
    h1 = _mm_relu_packed(f1p, None, W1, b1)
    f2p = _sc_acc(h1, idxp, dsqp)
    return _assemble(f1p, f2p, x)


# speedup vs baseline: 1.0125x; 1.0125x over previous
import functools

import jax
import jax.numpy as jnp
from jax import lax
from jax.experimental import pallas as pl
from jax.experimental.pallas import tpu as pltpu
from jax.experimental.pallas import tpu_sc as plsc

_N = 10000
_K = 16
_D = 256
_H = 256
_HW = _H // 2
_LANES = 16
_NTILES = 16
_CHUNK = 8
_CK = _CHUNK * _K
_NBUF = 4
_NP = 10240
_NPT0 = 512
_NPT1 = 128
_NC0 = _NPT0 // _CHUNK
_NC1 = _NPT1 // _CHUNK
_NE = _NP * _K + (_NPT0 - _NPT1) * _K
_NG2 = _H // (2 * _LANES)
_HIMASK = -65536


def _pack_words(v):
    f = v.shape[-1]
    lo = v[:, :f // 2].astype(jnp.bfloat16).astype(jnp.float32)
    hi = v[:, f // 2:].astype(jnp.bfloat16).astype(jnp.float32)
    lo_u = lax.shift_right_logical(
        lax.bitcast_convert_type(lo, jnp.int32), 16)
    hi_u = lax.bitcast_convert_type(hi, jnp.int32) & _HIMASK
    return lo_u | hi_u


def _unpack_words(w):
    lo = lax.bitcast_convert_type(lax.shift_left(w, 16), jnp.float32)
    hi = lax.bitcast_convert_type(w & _HIMASK, jnp.float32)
    return lo, hi


def _feat_full(fp):
    lo, hi = _unpack_words(fp)
    return jnp.concatenate([lo, hi], axis=1)


def _mm_relu_packed(a_packed, a_plain, feat_prev, w, b):
    if a_packed is not None:
        kd = a_packed.shape[1] * 2
    else:
        kd = a_plain.shape[1]
    hd = w.shape[1]
    bm = 1000

    def body(*refs):
        if a_packed is not None:
            a_ref, f_ref, w_ref, b_ref, o_ref = refs
            mean, mx = _unpack_words(a_ref[...])
            prev = _feat_full(f_ref[...])
            acc = jnp.dot(mean - prev, w_ref[:kd // 2],
                          preferred_element_type=jnp.float32)
            acc = acc + jnp.dot(mx - prev, w_ref[kd // 2:],
                                preferred_element_type=jnp.float32)
        else:
            a_ref, w_ref, b_ref, o_ref = refs
            acc = jnp.dot(a_ref[...], w_ref[...],
                          preferred_element_type=jnp.float32)
        o_ref[...] = _pack_words(jnp.maximum(acc + b_ref[...], 0.0))

    if a_packed is not None:
        args = (a_packed, feat_prev, w, b.reshape(1, hd))
        in_specs = [
            pl.BlockSpec((bm, a_packed.shape[1]), lambda i: (i, 0)),
            pl.BlockSpec((bm, feat_prev.shape[1]), lambda i: (i, 0)),
            pl.BlockSpec((kd, hd), lambda i: (0, 0)),
            pl.BlockSpec((1, hd), lambda i: (0, 0)),
        ]
    else:
        args = (a_plain, w, b.reshape(1, hd))
        in_specs = [
            pl.BlockSpec((bm, kd), lambda i: (i, 0)),
            pl.BlockSpec((kd, hd), lambda i: (0, 0)),
            pl.BlockSpec((1, hd), lambda i: (0, 0)),
        ]
    return pl.pallas_call(
        body,
        grid=(_N // bm,),
        in_specs=in_specs,
        out_specs=pl.BlockSpec((bm, hd // 2), lambda i: (i, 0)),
        out_shape=jax.ShapeDtypeStruct((_NP, hd // 2), jnp.int32),
    )(*args)


def _assemble(f1p, f0p, f2p, h1p, x):
    bm = 1000

    def body(f1_ref, f0_ref, f2_ref, h1_ref, x_ref, o_ref):
        m1, x1 = _unpack_words(f1_ref[...])
        m2, x2 = _unpack_words(f2_ref[...])
        p0 = _feat_full(f0_ref[...])
        p1 = _feat_full(h1_ref[...])
        o_ref[:, 0:_H] = m1 - p0
        o_ref[:, _H:2 * _H] = x1 - p0
        o_ref[:, 2 * _H:3 * _H] = m2 - p1
        o_ref[:, 3 * _H:4 * _H] = x2 - p1
        o_ref[:, 4 * _H:] = x_ref[...]

    return pl.pallas_call(
        body,
        grid=(_N // bm,),
        in_specs=[
            pl.BlockSpec((bm, _H), lambda i: (i, 0)),
            pl.BlockSpec((bm, _HW), lambda i: (i, 0)),
            pl.BlockSpec((bm, _H), lambda i: (i, 0)),
            pl.BlockSpec((bm, _HW), lambda i: (i, 0)),
            pl.BlockSpec((bm, _D), lambda i: (i, 0)),
        ],
        out_specs=pl.BlockSpec((bm, 4 * _H + _D), lambda i: (i, 0)),
        out_shape=jax.ShapeDtypeStruct((_N, 4 * _H + _D), jnp.float32),
    )(f1p, f0p, f2p, h1p, x)


def _tree(vals, op):
    while len(vals) > 1:
        vals = [op(vals[i], vals[i + 1]) for i in range(0, len(vals) - 1, 2)] \
            + ([vals[-1]] if len(vals) % 2 else [])
    return vals[0]


def _sc_acc_body(feat_hbm, idx_hbm, dsq_hbm, out_hbm, *scr):
    idx_all, w_all = scr[0], scr[1]
    rows = list(scr[2:2 + _NBUF])
    out = list(scr[2 + _NBUF:2 + 2 * _NBUF])
    gsem = list(scr[2 + 2 * _NBUF:2 + 3 * _NBUF])
    ssem = list(scr[2 + 3 * _NBUF:2 + 4 * _NBUF])

    cid = lax.axis_index("c")
    sid = lax.axis_index("s")
    base = jnp.where(cid == 0, sid * _NPT0,
                     _NTILES * _NPT0 + sid * _NPT1)
    base = pl.multiple_of(base, 16)
    nchunks = jnp.where(cid == 0, _NC0, _NC1)

    def gather_start(ci, b):
        pltpu.async_copy(
            feat_hbm.at[idx_all.at[pl.ds(ci * _CK, _CK)]], rows[b], gsem[b])

    def gather_wait(b):
        pltpu.make_async_copy(
            feat_hbm.at[idx_all.at[pl.ds(0, _CK)]], rows[b], gsem[b]).wait()

    def row0(ci):
        return pl.multiple_of(base + ci * _CHUNK, _CHUNK)

    def store_start(ci, b):
        pltpu.async_copy(
            out[b], out_hbm.at[pl.ds(row0(ci), _CHUNK)], ssem[b])

    def store_wait(b):
        pltpu.make_async_copy(
            out[b], out_hbm.at[pl.ds(0, _CHUNK)], ssem[b]).wait()

    pltpu.sync_copy(dsq_hbm.at[pl.ds(base * _K, _NPT0 * _K)], w_all)
    pltpu.sync_copy(idx_hbm.at[pl.ds(base * _K, _NPT0 * _K)], idx_all)
    for b in range(_NBUF):
        gather_start(b, b)

    def expbody(j, c):
        sl = pl.ds(j * _LANES, _LANES)
        w_all[sl] = jnp.exp(w_all[sl] * -10.0)
        return c

    lax.fori_loop(0, _NPT0 * _K // _LANES, expbody, 0)

    def compute(ci, b):
        rbuf = rows[b]
        ubuf = out[b]

        def node(n, c):
            wrow = w_all[pl.ds((ci * _CHUNK + n) * _K, _K)]
            dnums = lax.GatherDimensionNumbers(
                offset_dims=(), collapsed_slice_dims=(0,),
                start_index_map=(0,))
            wk = [lax.gather(wrow, jnp.full((_LANES, 1), k, jnp.int32),
                             dnums, slice_sizes=(1,),
                             mode=lax.GatherScatterMode.PROMISE_IN_BOUNDS)
                  for k in range(_K)]
            rbase = n * _K
            for g in range(_NG2):
                col = g * _LANES
                pa = []
                pb = []
                for k in range(_K):
                    va, vb = plsc.unpack(
                        plsc.bitcast(rbuf[rbase + k, pl.ds(col, _LANES)],
                                     jnp.bfloat16),
                        format=plsc.PackFormat.INTERLEAVED)
                    pa.append(va * wk[k])
                    pb.append(vb * wk[k])
                sa = _tree(pa, lambda x, y: x + y)
                sb = _tree(pb, lambda x, y: x + y)
                ma = _tree(pa, jnp.maximum)
                mb = _tree(pb, jnp.maximum)
                wlo = plsc.pack(sa * (1.0 / _K), ma,
                                format=plsc.PackFormat.INTERLEAVED)
                whi = plsc.pack(sb * (1.0 / _K), mb,
                                format=plsc.PackFormat.INTERLEAVED)
                ubuf[n, pl.ds(col, _LANES)] = plsc.bitcast(wlo, jnp.int32)
                ubuf[n, pl.ds(_HW + col, _LANES)] = plsc.bitcast(
                    whi, jnp.int32)
            return c

        lax.fori_loop(0, _CHUNK, node, 0)

    def group(i, c):
        for b in range(_NBUF):
            cch = i * _NBUF + b
            gather_wait(b)

            @pl.when(i > 0)
            def _():
                store_wait(b)

            compute(cch, b)

            @pl.when(cch + _NBUF < nchunks)
            def _():
                gather_start(cch + _NBUF, b)

            store_start(cch, b)
        return c

    lax.fori_loop(0, nchunks // _NBUF, group, 0)

    for b in range(_NBUF):
        store_wait(b)


_sc_acc = functools.partial(
    pl.kernel,
    out_type=jax.ShapeDtypeStruct((_NP, _H), jnp.int32),
    mesh=plsc.VectorSubcoreMesh(core_axis_name="c", subcore_axis_name="s",
                                num_cores=2, num_subcores=16),
    compiler_params=pltpu.CompilerParams(needs_layout_passes=False),
    scratch_types=(
        [pltpu.VMEM((_NPT0 * _K,), jnp.int32),
         pltpu.VMEM((_NPT0 * _K,), jnp.float32)]
        + [pltpu.VMEM((_CK, _HW), jnp.int32) for _ in range(_NBUF)]
        + [pltpu.VMEM((_CHUNK, _H), jnp.int32) for _ in range(_NBUF)]
        + [pltpu.SemaphoreType.DMA for _ in range(2 * _NBUF)]
    ),
)(_sc_acc_body)


def kernel(x, neighbor_indices, distancesq, W0, b0, W1, b1):
    idxp = jnp.pad(neighbor_indices.reshape(-1), (0, _NE - _N * _K))
    dsqp = jnp.pad(distancesq.reshape(-1), (0, _NE - _N * _K))
    f0 = _mm_relu_packed(None, x, None, W0, b0)
    f1p = _sc_acc(f0, idxp, dsqp)
    h1 = _mm_relu_packed(f1p, None, f0, W1, b1)
    f2p = _sc_acc(h1, idxp, dsqp)
    return _assemble(f1p, f0, f2p, h1, x)


# speedup vs baseline: 1.0274x; 1.0147x over previous
import functools

import jax
import jax.numpy as jnp
from jax import lax
from jax.experimental import pallas as pl
from jax.experimental.pallas import tpu as pltpu
from jax.experimental.pallas import tpu_sc as plsc

_N = 10000
_K = 16
_D = 256
_H = 256
_HW = _H // 2
_LANES = 16
_NTILES = 16
_CHUNK = 8
_CK = _CHUNK * _K
_NBUF = 4
_NP = 10240
_NPT0 = 544
_NPT1 = 96
_NC0 = _NPT0 // _CHUNK
_NC1 = _NPT1 // _CHUNK
_NE = _NP * _K + (_NPT0 - _NPT1) * _K
_NG2 = _H // (2 * _LANES)
_HIMASK = -65536


def _pack_words(v):
    f = v.shape[-1]
    lo = v[:, :f // 2].astype(jnp.bfloat16).astype(jnp.float32)
    hi = v[:, f // 2:].astype(jnp.bfloat16).astype(jnp.float32)
    lo_u = lax.shift_right_logical(
        lax.bitcast_convert_type(lo, jnp.int32), 16)
    hi_u = lax.bitcast_convert_type(hi, jnp.int32) & _HIMASK
    return lo_u | hi_u


def _unpack_words(w):
    lo = lax.bitcast_convert_type(lax.shift_left(w, 16), jnp.float32)
    hi = lax.bitcast_convert_type(w & _HIMASK, jnp.float32)
    return lo, hi


def _feat_full(fp):
    lo, hi = _unpack_words(fp)
    return jnp.concatenate([lo, hi], axis=1)


def _mm_relu_packed(a_packed, a_plain, feat_prev, w, b):
    if a_packed is not None:
        kd = a_packed.shape[1] * 2
    else:
        kd = a_plain.shape[1]
    hd = w.shape[1]
    bm = 1000

    def body(*refs):
        if a_packed is not None:
            a_ref, f_ref, w_ref, b_ref, o_ref = refs
            mean, mx = _unpack_words(a_ref[...])
            prev = _feat_full(f_ref[...])
            acc = jnp.dot(mean - prev, w_ref[:kd // 2],
                          preferred_element_type=jnp.float32)
            acc = acc + jnp.dot(mx - prev, w_ref[kd // 2:],
                                preferred_element_type=jnp.float32)
        else:
            a_ref, w_ref, b_ref, o_ref = refs
            acc = jnp.dot(a_ref[...], w_ref[...],
                          preferred_element_type=jnp.float32)
        o_ref[...] = _pack_words(jnp.maximum(acc + b_ref[...], 0.0))

    if a_packed is not None:
        args = (a_packed, feat_prev, w, b.reshape(1, hd))
        in_specs = [
            pl.BlockSpec((bm, a_packed.shape[1]), lambda i: (i, 0)),
            pl.BlockSpec((bm, feat_prev.shape[1]), lambda i: (i, 0)),
            pl.BlockSpec((kd, hd), lambda i: (0, 0)),
            pl.BlockSpec((1, hd), lambda i: (0, 0)),
        ]
    else:
        args = (a_plain, w, b.reshape(1, hd))
        in_specs = [
            pl.BlockSpec((bm, kd), lambda i: (i, 0)),
            pl.BlockSpec((kd, hd), lambda i: (0, 0)),
            pl.BlockSpec((1, hd), lambda i: (0, 0)),
        ]
    return pl.pallas_call(
        body,
        grid=(_N // bm,),
        in_specs=in_specs,
        out_specs=pl.BlockSpec((bm, hd // 2), lambda i: (i, 0)),
        out_shape=jax.ShapeDtypeStruct((_NP, hd // 2), jnp.int32),
    )(*args)


def _assemble(f1p, f0p, f2p, h1p, x):
    bm = 1000

    def body(f1_ref, f0_ref, f2_ref, h1_ref, x_ref, o_ref):
        m1, x1 = _unpack_words(f1_ref[...])
        m2, x2 = _unpack_words(f2_ref[...])
        p0 = _feat_full(f0_ref[...])
        p1 = _feat_full(h1_ref[...])
        o_ref[:, 0:_H] = m1 - p0
        o_ref[:, _H:2 * _H] = x1 - p0
        o_ref[:, 2 * _H:3 * _H] = m2 - p1
        o_ref[:, 3 * _H:4 * _H] = x2 - p1
        o_ref[:, 4 * _H:] = x_ref[...]

    return pl.pallas_call(
        body,
        grid=(_N // bm,),
        in_specs=[
            pl.BlockSpec((bm, _H), lambda i: (i, 0)),
            pl.BlockSpec((bm, _HW), lambda i: (i, 0)),
            pl.BlockSpec((bm, _H), lambda i: (i, 0)),
            pl.BlockSpec((bm, _HW), lambda i: (i, 0)),
            pl.BlockSpec((bm, _D), lambda i: (i, 0)),
        ],
        out_specs=pl.BlockSpec((bm, 4 * _H + _D), lambda i: (i, 0)),
        out_shape=jax.ShapeDtypeStruct((_N, 4 * _H + _D), jnp.float32),
    )(f1p, f0p, f2p, h1p, x)


def _tree(vals, op):
    while len(vals) > 1:
        vals = [op(vals[i], vals[i + 1]) for i in range(0, len(vals) - 1, 2)] \
            + ([vals[-1]] if len(vals) % 2 else [])
    return vals[0]


def _sc_acc_body(feat_hbm, idx_hbm, dsq_hbm, out_hbm, *scr):
    idx_all, w_all = scr[0], scr[1]
    rows = list(scr[2:2 + _NBUF])
    out = list(scr[2 + _NBUF:2 + 2 * _NBUF])
    gsem = list(scr[2 + 2 * _NBUF:2 + 3 * _NBUF])
    ssem = list(scr[2 + 3 * _NBUF:2 + 4 * _NBUF])

    cid = lax.axis_index("c")
    sid = lax.axis_index("s")
    base = jnp.where(cid == 0, sid * _NPT0,
                     _NTILES * _NPT0 + sid * _NPT1)
    base = pl.multiple_of(base, 16)
    nchunks = jnp.where(cid == 0, _NC0, _NC1)

    def gather_start(ci, b):
        pltpu.async_copy(
            feat_hbm.at[idx_all.at[pl.ds(ci * _CK, _CK)]], rows[b], gsem[b])

    def gather_wait(b):
        pltpu.make_async_copy(
            feat_hbm.at[idx_all.at[pl.ds(0, _CK)]], rows[b], gsem[b]).wait()

    def row0(ci):
        return pl.multiple_of(base + ci * _CHUNK, _CHUNK)

    def store_start(ci, b):
        pltpu.async_copy(
            out[b], out_hbm.at[pl.ds(row0(ci), _CHUNK)], ssem[b])

    def store_wait(b):
        pltpu.make_async_copy(
            out[b], out_hbm.at[pl.ds(0, _CHUNK)], ssem[b]).wait()

    pltpu.sync_copy(dsq_hbm.at[pl.ds(base * _K, _NPT0 * _K)], w_all)
    pltpu.sync_copy(idx_hbm.at[pl.ds(base * _K, _NPT0 * _K)], idx_all)
    for b in range(_NBUF):
        gather_start(b, b)

    def expbody(j, c):
        sl = pl.ds(j * _LANES, _LANES)
        w_all[sl] = jnp.exp(w_all[sl] * -10.0)
        return c

    lax.fori_loop(0, _NPT0 * _K // _LANES, expbody, 0)

    def compute(ci, b):
        rbuf = rows[b]
        ubuf = out[b]

        def node(n, c):
            wrow = w_all[pl.ds((ci * _CHUNK + n) * _K, _K)]
            dnums = lax.GatherDimensionNumbers(
                offset_dims=(), collapsed_slice_dims=(0,),
                start_index_map=(0,))
            wk = [lax.gather(wrow, jnp.full((_LANES, 1), k, jnp.int32),
                             dnums, slice_sizes=(1,),
                             mode=lax.GatherScatterMode.PROMISE_IN_BOUNDS)
                  for k in range(_K)]
            rbase = n * _K
            for g in range(_NG2):
                col = g * _LANES
                pa = []
                pb = []
                for k in range(_K):
                    va, vb = plsc.unpack(
                        plsc.bitcast(rbuf[rbase + k, pl.ds(col, _LANES)],
                                     jnp.bfloat16),
                        format=plsc.PackFormat.INTERLEAVED)
                    pa.append(va * wk[k])
                    pb.append(vb * wk[k])
                sa = _tree(pa, lambda x, y: x + y)
                sb = _tree(pb, lambda x, y: x + y)
                ma = _tree(pa, jnp.maximum)
                mb = _tree(pb, jnp.maximum)
                wlo = plsc.pack(sa * (1.0 / _K), ma,
                                format=plsc.PackFormat.INTERLEAVED)
                whi = plsc.pack(sb * (1.0 / _K), mb,
                                format=plsc.PackFormat.INTERLEAVED)
                ubuf[n, pl.ds(col, _LANES)] = plsc.bitcast(wlo, jnp.int32)
                ubuf[n, pl.ds(_HW + col, _LANES)] = plsc.bitcast(
                    whi, jnp.int32)
            return c

        lax.fori_loop(0, _CHUNK, node, 0)

    def group(i, c):
        for b in range(_NBUF):
            cch = i * _NBUF + b
            gather_wait(b)

            @pl.when(i > 0)
            def _():
                store_wait(b)

            compute(cch, b)

            @pl.when(cch + _NBUF < nchunks)
            def _():
                gather_start(cch + _NBUF, b)

            store_start(cch, b)
        return c

    lax.fori_loop(0, nchunks // _NBUF, group, 0)

    for b in range(_NBUF):
        store_wait(b)


_sc_acc = functools.partial(
    pl.kernel,
    out_type=jax.ShapeDtypeStruct((_NP, _H), jnp.int32),
    mesh=plsc.VectorSubcoreMesh(core_axis_name="c", subcore_axis_name="s",
                                num_cores=2, num_subcores=16),
    compiler_params=pltpu.CompilerParams(needs_layout_passes=False),
    scratch_types=(
        [pltpu.VMEM((_NPT0 * _K,), jnp.int32),
         pltpu.VMEM((_NPT0 * _K,), jnp.float32)]
        + [pltpu.VMEM((_CK, _HW), jnp.int32) for _ in range(_NBUF)]
        + [pltpu.VMEM((_CHUNK, _H), jnp.int32) for _ in range(_NBUF)]
        + [pltpu.SemaphoreType.DMA for _ in range(2 * _NBUF)]
    ),
)(_sc_acc_body)


def kernel(x, neighbor_indices, distancesq, W0, b0, W1, b1):
    idxp = jnp.pad(neighbor_indices.reshape(-1), (0, _NE - _N * _K))
    dsqp = jnp.pad(distancesq.reshape(-1), (0, _NE - _N * _K))
    f0 = _mm_relu_packed(None, x, None, W0, b0)
    f1p = _sc_acc(f0, idxp, dsqp)
    h1 = _mm_relu_packed(f1p, None, f0, W1, b1)
    f2p = _sc_acc(h1, idxp, dsqp)
    return _assemble(f1p, f0, f2p, h1, x)


# speedup vs baseline: 1.0798x; 1.0510x over previous
import functools

import jax
import jax.numpy as jnp
from jax import lax
from jax.experimental import pallas as pl
from jax.experimental.pallas import tpu as pltpu
from jax.experimental.pallas import tpu_sc as plsc

_N = 10000
_K = 16
_D = 256
_H = 256
_HW = _H // 2
_LANES = 16
_NTILES = 16
_CHUNK = 8
_CK = _CHUNK * _K
_NBUF = 4
_NP = 10240
_NPT0 = 576
_NPT1 = 64
_NC0 = _NPT0 // _CHUNK
_NC1 = _NPT1 // _CHUNK
_NE = _NP * _K + (_NPT0 - _NPT1) * _K
_NG2 = _H // (2 * _LANES)
_HIMASK = -65536


def _pack_words(v):
    f = v.shape[-1]
    lo = v[:, :f // 2].astype(jnp.bfloat16).astype(jnp.float32)
    hi = v[:, f // 2:].astype(jnp.bfloat16).astype(jnp.float32)
    lo_u = lax.shift_right_logical(
        lax.bitcast_convert_type(lo, jnp.int32), 16)
    hi_u = lax.bitcast_convert_type(hi, jnp.int32) & _HIMASK
    return lo_u | hi_u


def _unpack_words(w):
    lo = lax.bitcast_convert_type(lax.shift_left(w, 16), jnp.float32)
    hi = lax.bitcast_convert_type(w & _HIMASK, jnp.float32)
    return lo, hi


def _feat_full(fp):
    lo, hi = _unpack_words(fp)
    return jnp.concatenate([lo, hi], axis=1)


def _mm_relu_packed(a_packed, a_plain, feat_prev, w, b):
    if a_packed is not None:
        kd = a_packed.shape[1] * 2
    else:
        kd = a_plain.shape[1]
    hd = w.shape[1]
    bm = 1000

    def body(*refs):
        if a_packed is not None:
            a_ref, f_ref, w_ref, b_ref, o_ref = refs
            mean, mx = _unpack_words(a_ref[...])
            prev = _feat_full(f_ref[...])
            acc = jnp.dot(mean - prev, w_ref[:kd // 2],
                          preferred_element_type=jnp.float32)
            acc = acc + jnp.dot(mx - prev, w_ref[kd // 2:],
                                preferred_element_type=jnp.float32)
        else:
            a_ref, w_ref, b_ref, o_ref = refs
            acc = jnp.dot(a_ref[...], w_ref[...],
                          preferred_element_type=jnp.float32)
        o_ref[...] = _pack_words(jnp.maximum(acc + b_ref[...], 0.0))

    if a_packed is not None:
        args = (a_packed, feat_prev, w, b.reshape(1, hd))
        in_specs = [
            pl.BlockSpec((bm, a_packed.shape[1]), lambda i: (i, 0)),
            pl.BlockSpec((bm, feat_prev.shape[1]), lambda i: (i, 0)),
            pl.BlockSpec((kd, hd), lambda i: (0, 0)),
            pl.BlockSpec((1, hd), lambda i: (0, 0)),
        ]
    else:
        args = (a_plain, w, b.reshape(1, hd))
        in_specs = [
            pl.BlockSpec((bm, kd), lambda i: (i, 0)),
            pl.BlockSpec((kd, hd), lambda i: (0, 0)),
            pl.BlockSpec((1, hd), lambda i: (0, 0)),
        ]
    return pl.pallas_call(
        body,
        grid=(_N // bm,),
        in_specs=in_specs,
        out_specs=pl.BlockSpec((bm, hd // 2), lambda i: (i, 0)),
        out_shape=jax.ShapeDtypeStruct((_NP, hd // 2), jnp.int32),
    )(*args)


def _assemble(f1p, f0p, f2p, h1p, x):
    bm = 1000

    def body(f1_ref, f0_ref, f2_ref, h1_ref, x_ref, o_ref):
        m1, x1 = _unpack_words(f1_ref[...])
        m2, x2 = _unpack_words(f2_ref[...])
        p0 = _feat_full(f0_ref[...])
        p1 = _feat_full(h1_ref[...])
        o_ref[:, 0:_H] = m1 - p0
        o_ref[:, _H:2 * _H] = x1 - p0
        o_ref[:, 2 * _H:3 * _H] = m2 - p1
        o_ref[:, 3 * _H:4 * _H] = x2 - p1
        o_ref[:, 4 * _H:] = x_ref[...]

    return pl.pallas_call(
        body,
        grid=(_N // bm,),
        in_specs=[
            pl.BlockSpec((bm, _H), lambda i: (i, 0)),
            pl.BlockSpec((bm, _HW), lambda i: (i, 0)),
            pl.BlockSpec((bm, _H), lambda i: (i, 0)),
            pl.BlockSpec((bm, _HW), lambda i: (i, 0)),
            pl.BlockSpec((bm, _D), lambda i: (i, 0)),
        ],
        out_specs=pl.BlockSpec((bm, 4 * _H + _D), lambda i: (i, 0)),
        out_shape=jax.ShapeDtypeStruct((_N, 4 * _H + _D), jnp.float32),
    )(f1p, f0p, f2p, h1p, x)


def _tree(vals, op):
    while len(vals) > 1:
        vals = [op(vals[i], vals[i + 1]) for i in range(0, len(vals) - 1, 2)] \
            + ([vals[-1]] if len(vals) % 2 else [])
    return vals[0]


def _sc_acc_body(feat_hbm, idx_hbm, dsq_hbm, out_hbm, *scr):
    idx_all, w_all = scr[0], scr[1]
    rows = list(scr[2:2 + _NBUF])
    out = list(scr[2 + _NBUF:2 + 2 * _NBUF])
    gsem = list(scr[2 + 2 * _NBUF:2 + 3 * _NBUF])
    ssem = list(scr[2 + 3 * _NBUF:2 + 4 * _NBUF])

    cid = lax.axis_index("c")
    sid = lax.axis_index("s")
    base = jnp.where(cid == 0, sid * _NPT0,
                     _NTILES * _NPT0 + sid * _NPT1)
    base = pl.multiple_of(base, 16)
    nchunks = jnp.where(cid == 0, _NC0, _NC1)

    def gather_start(ci, b):
        pltpu.async_copy(
            feat_hbm.at[idx_all.at[pl.ds(ci * _CK, _CK)]], rows[b], gsem[b])

    def gather_wait(b):
        pltpu.make_async_copy(
            feat_hbm.at[idx_all.at[pl.ds(0, _CK)]], rows[b], gsem[b]).wait()

    def row0(ci):
        return pl.multiple_of(base + ci * _CHUNK, _CHUNK)

    def store_start(ci, b):
        pltpu.async_copy(
            out[b], out_hbm.at[pl.ds(row0(ci), _CHUNK)], ssem[b])

    def store_wait(b):
        pltpu.make_async_copy(
            out[b], out_hbm.at[pl.ds(0, _CHUNK)], ssem[b]).wait()

    pltpu.sync_copy(dsq_hbm.at[pl.ds(base * _K, _NPT0 * _K)], w_all)
    pltpu.sync_copy(idx_hbm.at[pl.ds(base * _K, _NPT0 * _K)], idx_all)
    for b in range(_NBUF):
        gather_start(b, b)

    def expbody(j, c):
        sl = pl.ds(j * _LANES, _LANES)
        w_all[sl] = jnp.exp(w_all[sl] * -10.0)
        return c

    lax.fori_loop(0, _NPT0 * _K // _LANES, expbody, 0)

    def compute(ci, b):
        rbuf = rows[b]
        ubuf = out[b]

        def node(n, c):
            wrow = w_all[pl.ds((ci * _CHUNK + n) * _K, _K)]
            dnums = lax.GatherDimensionNumbers(
                offset_dims=(), collapsed_slice_dims=(0,),
                start_index_map=(0,))
            wk = [lax.gather(wrow, jnp.full((_LANES, 1), k, jnp.int32),
                             dnums, slice_sizes=(1,),
                             mode=lax.GatherScatterMode.PROMISE_IN_BOUNDS)
                  for k in range(_K)]
            rbase = n * _K
            for g in range(_NG2):
                col = g * _LANES
                pa = []
                pb = []
                for k in range(_K):
                    va, vb = plsc.unpack(
                        plsc.bitcast(rbuf[rbase + k, pl.ds(col, _LANES)],
                                     jnp.bfloat16),
                        format=plsc.PackFormat.INTERLEAVED)
                    pa.append(va * wk[k])
                    pb.append(vb * wk[k])
                sa = _tree(pa, lambda x, y: x + y)
                sb = _tree(pb, lambda x, y: x + y)
                ma = _tree(pa, jnp.maximum)
                mb = _tree(pb, jnp.maximum)
                wlo = plsc.pack(sa * (1.0 / _K), ma,
                                format=plsc.PackFormat.INTERLEAVED)
                whi = plsc.pack(sb * (1.0 / _K), mb,
                                format=plsc.PackFormat.INTERLEAVED)
                ubuf[n, pl.ds(col, _LANES)] = plsc.bitcast(wlo, jnp.int32)
                ubuf[n, pl.ds(_HW + col, _LANES)] = plsc.bitcast(
                    whi, jnp.int32)
            return c

        lax.fori_loop(0, _CHUNK, node, 0)

    def group(i, c):
        for b in range(_NBUF):
            cch = i * _NBUF + b
            gather_wait(b)

            @pl.when(i > 0)
            def _():
                store_wait(b)

            compute(cch, b)

            @pl.when(cch + _NBUF < nchunks)
            def _():
                gather_start(cch + _NBUF, b)

            store_start(cch, b)
        return c

    lax.fori_loop(0, nchunks // _NBUF, group, 0)

    for b in range(_NBUF):
        store_wait(b)


_sc_acc = functools.partial(
    pl.kernel,
    out_type=jax.ShapeDtypeStruct((_NP, _H), jnp.int32),
    mesh=plsc.VectorSubcoreMesh(core_axis_name="c", subcore_axis_name="s",
                                num_cores=2, num_subcores=16),
    compiler_params=pltpu.CompilerParams(needs_layout_passes=False),
    scratch_types=(
        [pltpu.VMEM((_NPT0 * _K,), jnp.int32),
         pltpu.VMEM((_NPT0 * _K,), jnp.float32)]
        + [pltpu.VMEM((_CK, _HW), jnp.int32) for _ in range(_NBUF)]
        + [pltpu.VMEM((_CHUNK, _H), jnp.int32) for _ in range(_NBUF)]
        + [pltpu.SemaphoreType.DMA for _ in range(2 * _NBUF)]
    ),
)(_sc_acc_body)


def kernel(x, neighbor_indices, distancesq, W0, b0, W1, b1):
    idxp = jnp.pad(neighbor_indices.reshape(-1), (0, _NE - _N * _K))
    dsqp = jnp.pad(distancesq.reshape(-1), (0, _NE - _N * _K))
    f0 = _mm_relu_packed(None, x, None, W0, b0)
    f1p = _sc_acc(f0, idxp, dsqp)
    h1 = _mm_relu_packed(f1p, None, f0, W1, b1)
    f2p = _sc_acc(h1, idxp, dsqp)
    return _assemble(f1p, f0, f2p, h1, x)
